# Initial kernel scaffold; baseline (speedup 1.0000x reference)
import jax
import jax.numpy as jnp
from jax.experimental import pallas as pl


def kernel(xyz, table, W0, W1, W2, xyz_bound_min, xyz_bound_max):
    raise NotImplementedError("write your pallas kernel here")


# speedup vs baseline: 45.9779x; 45.9779x over previous
import functools

import jax
import jax.numpy as jnp
import numpy as np
from jax import lax
from jax.experimental import pallas as pl
from jax.experimental.pallas import tpu as pltpu
from jax.experimental.pallas import tpu_sc as plsc

N_POINTS = 262144
N_LEVELS = 16
BASE_RES = 16
TABLE_SIZE = 2 ** 15
D_HIDDEN = 64
D_OUT = 8

P1 = -1640531535
P2 = 805459861

CHUNK = 4096
BN = 512


def _encode_body(tabs, x_h, y_h, z_h, bnds, out, tab_v, x_v, y_v, z_v,
                 f0_v, f1_v, m_v, b_v):
    c = lax.axis_index("c")
    s = lax.axis_index("s")
    n = x_h.shape[0]
    is_mask_tile = jnp.logical_and(c == 0, s == 0)

    pltpu.sync_copy(tabs.at[pl.ds((c * 16 + s) * (2 * TABLE_SIZE),
                                  2 * TABLE_SIZE)], tab_v)
    pltpu.sync_copy(bnds, b_v)

    mnx = b_v[pl.ds(0, 16)]
    mny = b_v[pl.ds(16, 16)]
    mnz = b_v[pl.ds(32, 16)]
    rgx = b_v[pl.ds(48, 16)]
    rgy = b_v[pl.ds(64, 16)]
    rgz = b_v[pl.ds(80, 16)]

    res_i = lax.shift_left(jnp.int32(BASE_RES), s)
    resv = jnp.broadcast_to(res_i, (16,)).astype(jnp.float32)

    def chunk_body(k, _):
        base = k * CHUNK
        pltpu.sync_copy(x_h.at[pl.ds(base, CHUNK)], x_v)
        pltpu.sync_copy(y_h.at[pl.ds(base, CHUNK)], y_v)
        pltpu.sync_copy(z_h.at[pl.ds(base, CHUNK)], z_v)

        def vec_body(i, _):
            off = i * 16
            x01 = (x_v[pl.ds(off, 16)] - mnx) / rgx
            y01 = (y_v[pl.ds(off, 16)] - mny) / rgy
            z01 = (z_v[pl.ds(off, 16)] - mnz) / rgz
            px = x01 * resv
            py = y01 * resv
            pz = z01 * resv
            pxi = px.astype(jnp.int32)
            pyi = py.astype(jnp.int32)
            pzi = pz.astype(jnp.int32)
            fx = px - pxi.astype(jnp.float32)
            fy = py - pyi.astype(jnp.float32)
            fz = pz - pzi.astype(jnp.float32)
            gx = 1.0 - fx
            gy = 1.0 - fy
            gz = 1.0 - fz
            hx = [pxi, pxi + 1]
            hy0 = pyi * P1
            hy = [hy0, hy0 + P1]
            hz0 = pzi * P2
            hz = [hz0, hz0 + P2]
            wxy = [gx * gy, fx * gy, gx * fy, fx * fy]
            wz = [gz, fz]
            acc0 = jnp.zeros((16,), jnp.float32)
            acc1 = jnp.zeros((16,), jnp.float32)
            for corner in range(8):
                bx = corner & 1
                by = (corner >> 1) & 1
                bz = corner >> 2
                h = hx[bx] ^ hy[by] ^ hz[bz]
                idx2 = lax.shift_left(h & (TABLE_SIZE - 1), 1)
                v0 = plsc.load_gather(tab_v, [idx2])
                v1 = plsc.load_gather(tab_v, [idx2 | 1])
                w = wxy[bx + 2 * by] * wz[bz]
                acc0 = acc0 + w * v0
                acc1 = acc1 + w * v1
            f0_v[pl.ds(off, 16)] = acc0
            f1_v[pl.ds(off, 16)] = acc1

            @pl.when(is_mask_tile)
            def _():
                inb = ((x01 >= 0.0) & (x01 <= 1.0)
                       & (y01 >= 0.0) & (y01 <= 1.0)
                       & (z01 >= 0.0) & (z01 <= 1.0))
                m_v[pl.ds(off, 16)] = jnp.where(inb, jnp.float32(1.0),
                                                jnp.float32(0.0))
            return None

        lax.fori_loop(0, CHUNK // 16, vec_body, None)

        r0 = 4 * s + 2 * c
        pltpu.sync_copy(f0_v, out.at[pl.ds(r0 * n + base, CHUNK)])
        pltpu.sync_copy(f1_v, out.at[pl.ds((r0 + 1) * n + base, CHUNK)])

        @pl.when(is_mask_tile)
        def _():
            pltpu.sync_copy(m_v, out.at[pl.ds(64 * n + base, CHUNK)])
        return None

    lax.fori_loop(0, N_POINTS // CHUNK, chunk_body, None)


def _mlp_body(enc_ref, w0_ref, w1_ref, w2_ref, out_ref):
    e = enc_ref[...]
    enc = e[0:64]
    maskf = e[64:65]
    w0 = w0_ref[...]
    w1 = w1_ref[...]
    w2 = w2_ref[...]
    dn = (((0,), (0,)), ((), ()))
    h = jnp.maximum(
        lax.dot_general(w0, enc, dn, preferred_element_type=jnp.float32), 0.0)
    h = jnp.maximum(
        lax.dot_general(w1, h, dn, preferred_element_type=jnp.float32), 0.0)
    r = lax.dot_general(w2, h, dn, preferred_element_type=jnp.float32)
    m = maskf > 0.5
    dxyz = jnp.where(m, r[0:3], 0.0)
    rot0 = jnp.where(m, r[3:4], 1.0)
    rot123 = jnp.where(m, r[4:7], 0.0)
    out_ref[...] = jnp.concatenate([dxyz, rot0, rot123, maskf], axis=0)


def kernel(xyz, table, W0, W1, W2, xyz_bound_min, xyz_bound_max):
    n = xyz.shape[0]
    x_h = xyz[:, 0]
    y_h = xyz[:, 1]
    z_h = xyz[:, 2]
    tabs = (table.reshape(N_LEVELS, TABLE_SIZE, 2, 2)
            .transpose(2, 0, 1, 3)
            .reshape(2 * N_LEVELS * 2 * TABLE_SIZE))
    rng = xyz_bound_max - xyz_bound_min
    bnds = jnp.concatenate([
        jnp.broadcast_to(xyz_bound_min[:, None], (3, 16)),
        jnp.broadcast_to(rng[:, None], (3, 16)),
    ]).reshape(96)

    mesh = plsc.VectorSubcoreMesh(core_axis_name="c", subcore_axis_name="s",
                                  num_cores=2, num_subcores=16)
    encode = functools.partial(
        pl.kernel,
        out_type=jax.ShapeDtypeStruct((72 * n,), jnp.float32),
        mesh=mesh,
        compiler_params=pltpu.CompilerParams(needs_layout_passes=False),
        scratch_types=[
            pltpu.VMEM((2 * TABLE_SIZE,), jnp.float32),
            pltpu.VMEM((CHUNK,), jnp.float32),
            pltpu.VMEM((CHUNK,), jnp.float32),
            pltpu.VMEM((CHUNK,), jnp.float32),
            pltpu.VMEM((CHUNK,), jnp.float32),
            pltpu.VMEM((CHUNK,), jnp.float32),
            pltpu.VMEM((CHUNK,), jnp.float32),
            pltpu.VMEM((96,), jnp.float32),
        ],
    )(_encode_body)
    enc = encode(tabs, x_h, y_h, z_h, bnds).reshape(72, n)

    out8 = pl.pallas_call(
        _mlp_body,
        grid=(n // BN,),
        in_specs=[
            pl.BlockSpec((72, BN), lambda i: (0, i)),
            pl.BlockSpec((64, 64), lambda i: (0, 0)),
            pl.BlockSpec((64, 64), lambda i: (0, 0)),
            pl.BlockSpec((64, 8), lambda i: (0, 0)),
        ],
        out_specs=pl.BlockSpec((8, BN), lambda i: (0, i)),
        out_shape=jax.ShapeDtypeStruct((8, n), jnp.float32),
    )(enc, W0, W1, W2)

    mask = out8[7] > 0.0
    d_xyz = out8[0:3].T
    d_rot = out8[3:7].T
    return (mask, d_xyz, d_rot)


# speedup vs baseline: 91.3721x; 1.9873x over previous
import functools

import jax
import jax.numpy as jnp
import numpy as np
from jax import lax
from jax.experimental import pallas as pl
from jax.experimental.pallas import tpu as pltpu
from jax.experimental.pallas import tpu_sc as plsc

N_LEVELS = 16
BASE_RES = 16
TABLE_SIZE = 2 ** 15

P1 = -1640531535
P2 = 805459861

CHUNK = 4096
BN = 512


def _encode_body(tabs, x_h, y_h, z_h, bnds, out, tab_v, x_v, y_v, z_v,
                 f0_v, f1_v, f2_v, f3_v, m_v, b_v):
    c = lax.axis_index("c")
    s = lax.axis_index("s")
    n = x_h.shape[0]
    half_n = n // 2
    is_mask_tile = s == 0

    pltpu.sync_copy(tabs.at[pl.ds(s * (2 * TABLE_SIZE), 2 * TABLE_SIZE)],
                    tab_v)
    pltpu.sync_copy(bnds, b_v)

    mnx = b_v[pl.ds(0, 16)]
    mny = b_v[pl.ds(16, 16)]
    mnz = b_v[pl.ds(32, 16)]
    rgx = b_v[pl.ds(48, 16)]
    rgy = b_v[pl.ds(64, 16)]
    rgz = b_v[pl.ds(80, 16)]

    res_i = lax.shift_left(jnp.int32(BASE_RES), s)
    resv = jnp.broadcast_to(res_i, (16,)).astype(jnp.float32)
    hi16 = jnp.int32(-65536)

    def chunk_body(k, _):
        base = c * half_n + k * CHUNK
        pltpu.sync_copy(x_h.at[pl.ds(base, CHUNK)], x_v)
        pltpu.sync_copy(y_h.at[pl.ds(base, CHUNK)], y_v)
        pltpu.sync_copy(z_h.at[pl.ds(base, CHUNK)], z_v)

        def vec_body(i, _):
            off = i * 16
            x01 = (x_v[pl.ds(off, 16)] - mnx) / rgx
            y01 = (y_v[pl.ds(off, 16)] - mny) / rgy
            z01 = (z_v[pl.ds(off, 16)] - mnz) / rgz
            px = x01 * resv
            py = y01 * resv
            pz = z01 * resv
            pxi = px.astype(jnp.int32)
            pyi = py.astype(jnp.int32)
            pzi = pz.astype(jnp.int32)
            fx = px - pxi.astype(jnp.float32)
            fy = py - pyi.astype(jnp.float32)
            fz = pz - pzi.astype(jnp.float32)
            gx = 1.0 - fx
            gy = 1.0 - fy
            gz = 1.0 - fz
            hx = [pxi, pxi + 1]
            hy0 = pyi * P1
            hy = [hy0, hy0 + P1]
            hz0 = pzi * P2
            hz = [hz0, hz0 + P2]
            wxy = [gx * gy, fx * gy, gx * fy, fx * fy]
            wz = [gz, fz]
            acc = [jnp.zeros((16,), jnp.float32) for _ in range(4)]
            for corner in range(8):
                bx = corner & 1
                by = (corner >> 1) & 1
                bz = corner >> 2
                h = hx[bx] ^ hy[by] ^ hz[bz]
                idx2 = lax.shift_left(h & (TABLE_SIZE - 1), 1)
                w0 = plsc.load_gather(tab_v, [idx2])
                w1 = plsc.load_gather(tab_v, [idx2 | 1])
                a0 = plsc.bitcast(lax.shift_left(w0, 16), jnp.float32)
                a1 = plsc.bitcast(w0 & hi16, jnp.float32)
                a2 = plsc.bitcast(lax.shift_left(w1, 16), jnp.float32)
                a3 = plsc.bitcast(w1 & hi16, jnp.float32)
                w = wxy[bx + 2 * by] * wz[bz]
                acc[0] = acc[0] + w * a0
                acc[1] = acc[1] + w * a1
                acc[2] = acc[2] + w * a2
                acc[3] = acc[3] + w * a3
            f0_v[pl.ds(off, 16)] = acc[0]
            f1_v[pl.ds(off, 16)] = acc[1]
            f2_v[pl.ds(off, 16)] = acc[2]
            f3_v[pl.ds(off, 16)] = acc[3]

            @pl.when(is_mask_tile)
            def _():
                inb = ((x01 >= 0.0) & (x01 <= 1.0)
                       & (y01 >= 0.0) & (y01 <= 1.0)
                       & (z01 >= 0.0) & (z01 <= 1.0))
                m_v[pl.ds(off, 16)] = jnp.where(inb, jnp.float32(1.0),
                                                jnp.float32(0.0))
            return None

        lax.fori_loop(0, CHUNK // 16, vec_body, None)

        r0 = 4 * s
        pltpu.sync_copy(f0_v, out.at[pl.ds(r0 * n + base, CHUNK)])
        pltpu.sync_copy(f1_v, out.at[pl.ds((r0 + 1) * n + base, CHUNK)])
        pltpu.sync_copy(f2_v, out.at[pl.ds((r0 + 2) * n + base, CHUNK)])
        pltpu.sync_copy(f3_v, out.at[pl.ds((r0 + 3) * n + base, CHUNK)])

        @pl.when(is_mask_tile)
        def _():
            pltpu.sync_copy(m_v, out.at[pl.ds(64 * n + base, CHUNK)])
        return None

    lax.fori_loop(0, half_n // CHUNK, chunk_body, None)


def _mlp_body(enc_ref, w0_ref, w1_ref, w2_ref, out_ref):
    e = enc_ref[...]
    enc = e[0:64]
    maskf = e[64:65]
    w0 = w0_ref[...]
    w1 = w1_ref[...]
    w2 = w2_ref[...]
    dn = (((0,), (0,)), ((), ()))
    h = jnp.maximum(
        lax.dot_general(w0, enc, dn, preferred_element_type=jnp.float32), 0.0)
    h = jnp.maximum(
        lax.dot_general(w1, h, dn, preferred_element_type=jnp.float32), 0.0)
    r = lax.dot_general(w2, h, dn, preferred_element_type=jnp.float32)
    m = maskf > 0.5
    dxyz = jnp.where(m, r[0:3], 0.0)
    rot0 = jnp.where(m, r[3:4], 1.0)
    rot123 = jnp.where(m, r[4:7], 0.0)
    out_ref[...] = jnp.concatenate([dxyz, rot0, rot123, maskf], axis=0)


def kernel(xyz, table, W0, W1, W2, xyz_bound_min, xyz_bound_max):
    n = xyz.shape[0]
    x_h = xyz[:, 0]
    y_h = xyz[:, 1]
    z_h = xyz[:, 2]
    tb = table.astype(jnp.bfloat16)
    tabs = jax.lax.bitcast_convert_type(
        tb.reshape(N_LEVELS, TABLE_SIZE, 2, 2), jnp.int32
    ).reshape(N_LEVELS * TABLE_SIZE * 2)
    rng = xyz_bound_max - xyz_bound_min
    bnds = jnp.concatenate([
        jnp.broadcast_to(xyz_bound_min[:, None], (3, 16)),
        jnp.broadcast_to(rng[:, None], (3, 16)),
    ]).reshape(96)

    mesh = plsc.VectorSubcoreMesh(core_axis_name="c", subcore_axis_name="s",
                                  num_cores=2, num_subcores=16)
    encode = functools.partial(
        pl.kernel,
        out_type=jax.ShapeDtypeStruct((72 * n,), jnp.float32),
        mesh=mesh,
        compiler_params=pltpu.CompilerParams(needs_layout_passes=False),
        scratch_types=[
            pltpu.VMEM((2 * TABLE_SIZE,), jnp.int32),
            pltpu.VMEM((CHUNK,), jnp.float32),
            pltpu.VMEM((CHUNK,), jnp.float32),
            pltpu.VMEM((CHUNK,), jnp.float32),
            pltpu.VMEM((CHUNK,), jnp.float32),
            pltpu.VMEM((CHUNK,), jnp.float32),
            pltpu.VMEM((CHUNK,), jnp.float32),
            pltpu.VMEM((CHUNK,), jnp.float32),
            pltpu.VMEM((CHUNK,), jnp.float32),
            pltpu.VMEM((96,), jnp.float32),
        ],
    )(_encode_body)
    enc = encode(tabs, x_h, y_h, z_h, bnds).reshape(72, n)

    out8 = pl.pallas_call(
        _mlp_body,
        grid=(n // BN,),
        in_specs=[
            pl.BlockSpec((72, BN), lambda i: (0, i)),
            pl.BlockSpec((64, 64), lambda i: (0, 0)),
            pl.BlockSpec((64, 64), lambda i: (0, 0)),
            pl.BlockSpec((64, 8), lambda i: (0, 0)),
        ],
        out_specs=pl.BlockSpec((8, BN), lambda i: (0, i)),
        out_shape=jax.ShapeDtypeStruct((8, n), jnp.float32),
    )(enc, W0, W1, W2)

    mask = out8[7] > 0.0
    d_xyz = out8[0:3].T
    d_rot = out8[3:7].T
    return (mask, d_xyz, d_rot)


# speedup vs baseline: 117.2889x; 1.2836x over previous
import functools

import jax
import jax.numpy as jnp
import numpy as np
from jax import lax
from jax.experimental import pallas as pl
from jax.experimental.pallas import tpu as pltpu
from jax.experimental.pallas import tpu_sc as plsc

N_LEVELS = 16
BASE_RES = 16
TABLE_SIZE = 2 ** 15

P1 = -1640531535
P2 = 805459861

CHUNK = 4096
BN = 4096


def _encode_body(tabs, x_h, y_h, z_h, bnds, out, tab_v, x_v, y_v, z_v,
                 f0_v, f1_v, f2_v, f3_v, m_v, b_v):
    c = lax.axis_index("c")
    s = lax.axis_index("s")
    n = x_h.shape[0]
    half_n = n // 2
    is_mask_tile = s == 0

    pltpu.sync_copy(tabs.at[pl.ds(s * (2 * TABLE_SIZE), 2 * TABLE_SIZE)],
                    tab_v)
    pltpu.sync_copy(bnds, b_v)

    mnx = b_v[pl.ds(0, 16)]
    mny = b_v[pl.ds(16, 16)]
    mnz = b_v[pl.ds(32, 16)]
    rgx = b_v[pl.ds(48, 16)]
    rgy = b_v[pl.ds(64, 16)]
    rgz = b_v[pl.ds(80, 16)]

    res_i = lax.shift_left(jnp.int32(BASE_RES), s)
    resv = jnp.broadcast_to(res_i, (16,)).astype(jnp.float32)
    hi16 = jnp.int32(-65536)

    def chunk_body(k, _):
        base = c * half_n + k * CHUNK
        pltpu.sync_copy(x_h.at[pl.ds(base, CHUNK)], x_v)
        pltpu.sync_copy(y_h.at[pl.ds(base, CHUNK)], y_v)
        pltpu.sync_copy(z_h.at[pl.ds(base, CHUNK)], z_v)

        def vec_body(i, _):
            off = i * 16
            x01 = (x_v[pl.ds(off, 16)] - mnx) / rgx
            y01 = (y_v[pl.ds(off, 16)] - mny) / rgy
            z01 = (z_v[pl.ds(off, 16)] - mnz) / rgz
            px = x01 * resv
            py = y01 * resv
            pz = z01 * resv
            pxi = px.astype(jnp.int32)
            pyi = py.astype(jnp.int32)
            pzi = pz.astype(jnp.int32)
            fx = px - pxi.astype(jnp.float32)
            fy = py - pyi.astype(jnp.float32)
            fz = pz - pzi.astype(jnp.float32)
            gx = 1.0 - fx
            gy = 1.0 - fy
            gz = 1.0 - fz
            hx = [pxi, pxi + 1]
            hy0 = pyi * P1
            hy = [hy0, hy0 + P1]
            hz0 = pzi * P2
            hz = [hz0, hz0 + P2]
            wxy = [gx * gy, fx * gy, gx * fy, fx * fy]
            wz = [gz, fz]
            acc = [jnp.zeros((16,), jnp.float32) for _ in range(4)]
            for corner in range(8):
                bx = corner & 1
                by = (corner >> 1) & 1
                bz = corner >> 2
                h = hx[bx] ^ hy[by] ^ hz[bz]
                idx2 = lax.shift_left(h & (TABLE_SIZE - 1), 1)
                w0 = plsc.load_gather(tab_v, [idx2])
                w1 = plsc.load_gather(tab_v, [idx2 | 1])
                a0 = plsc.bitcast(lax.shift_left(w0, 16), jnp.float32)
                a1 = plsc.bitcast(w0 & hi16, jnp.float32)
                a2 = plsc.bitcast(lax.shift_left(w1, 16), jnp.float32)
                a3 = plsc.bitcast(w1 & hi16, jnp.float32)
                w = wxy[bx + 2 * by] * wz[bz]
                acc[0] = acc[0] + w * a0
                acc[1] = acc[1] + w * a1
                acc[2] = acc[2] + w * a2
                acc[3] = acc[3] + w * a3
            f0_v[pl.ds(off, 16)] = acc[0]
            f1_v[pl.ds(off, 16)] = acc[1]
            f2_v[pl.ds(off, 16)] = acc[2]
            f3_v[pl.ds(off, 16)] = acc[3]

            @pl.when(is_mask_tile)
            def _():
                inb = ((x01 >= 0.0) & (x01 <= 1.0)
                       & (y01 >= 0.0) & (y01 <= 1.0)
                       & (z01 >= 0.0) & (z01 <= 1.0))
                m_v[pl.ds(off, 16)] = jnp.where(inb, jnp.float32(1.0),
                                                jnp.float32(0.0))
            return None

        lax.fori_loop(0, CHUNK // 16, vec_body, None)

        r0 = 4 * s
        pltpu.sync_copy(f0_v, out.at[pl.ds(r0 * n + base, CHUNK)])
        pltpu.sync_copy(f1_v, out.at[pl.ds((r0 + 1) * n + base, CHUNK)])
        pltpu.sync_copy(f2_v, out.at[pl.ds((r0 + 2) * n + base, CHUNK)])
        pltpu.sync_copy(f3_v, out.at[pl.ds((r0 + 3) * n + base, CHUNK)])

        @pl.when(is_mask_tile)
        def _():
            pltpu.sync_copy(m_v, out.at[pl.ds(64 * n + base, CHUNK)])
        return None

    lax.fori_loop(0, half_n // CHUNK, chunk_body, None)


def _mlp_body(enc_ref, w0_ref, w1_ref, w2_ref, out_ref):
    e = enc_ref[...]
    enc = e[0:64]
    maskf = e[64:65]
    w0 = w0_ref[...]
    w1 = w1_ref[...]
    w2 = w2_ref[...]
    dn = (((0,), (0,)), ((), ()))
    h = jnp.maximum(
        lax.dot_general(w0, enc, dn, preferred_element_type=jnp.float32), 0.0)
    h = jnp.maximum(
        lax.dot_general(w1, h, dn, preferred_element_type=jnp.float32), 0.0)
    r = lax.dot_general(w2, h, dn, preferred_element_type=jnp.float32)
    m = maskf > 0.5
    dxyz = jnp.where(m, r[0:3], 0.0)
    rot0 = jnp.where(m, r[3:4], 1.0)
    rot123 = jnp.where(m, r[4:7], 0.0)
    out_ref[...] = jnp.concatenate([dxyz, rot0, rot123, maskf], axis=0)


def kernel(xyz, table, W0, W1, W2, xyz_bound_min, xyz_bound_max):
    n = xyz.shape[0]
    x_h = xyz[:, 0]
    y_h = xyz[:, 1]
    z_h = xyz[:, 2]
    tb = table.astype(jnp.bfloat16)
    tabs = jax.lax.bitcast_convert_type(
        tb.reshape(N_LEVELS, TABLE_SIZE, 2, 2), jnp.int32
    ).reshape(N_LEVELS * TABLE_SIZE * 2)
    rng = xyz_bound_max - xyz_bound_min
    bnds = jnp.concatenate([
        jnp.broadcast_to(xyz_bound_min[:, None], (3, 16)),
        jnp.broadcast_to(rng[:, None], (3, 16)),
    ]).reshape(96)

    mesh = plsc.VectorSubcoreMesh(core_axis_name="c", subcore_axis_name="s",
                                  num_cores=2, num_subcores=16)
    encode = functools.partial(
        pl.kernel,
        out_type=jax.ShapeDtypeStruct((72 * n,), jnp.float32),
        mesh=mesh,
        compiler_params=pltpu.CompilerParams(needs_layout_passes=False),
        scratch_types=[
            pltpu.VMEM((2 * TABLE_SIZE,), jnp.int32),
            pltpu.VMEM((CHUNK,), jnp.float32),
            pltpu.VMEM((CHUNK,), jnp.float32),
            pltpu.VMEM((CHUNK,), jnp.float32),
            pltpu.VMEM((CHUNK,), jnp.float32),
            pltpu.VMEM((CHUNK,), jnp.float32),
            pltpu.VMEM((CHUNK,), jnp.float32),
            pltpu.VMEM((CHUNK,), jnp.float32),
            pltpu.VMEM((CHUNK,), jnp.float32),
            pltpu.VMEM((96,), jnp.float32),
        ],
    )(_encode_body)
    enc = encode(tabs, x_h, y_h, z_h, bnds).reshape(72, n)

    out8 = pl.pallas_call(
        _mlp_body,
        grid=(n // BN,),
        in_specs=[
            pl.BlockSpec((72, BN), lambda i: (0, i)),
            pl.BlockSpec((64, 64), lambda i: (0, 0)),
            pl.BlockSpec((64, 64), lambda i: (0, 0)),
            pl.BlockSpec((64, 8), lambda i: (0, 0)),
        ],
        out_specs=pl.BlockSpec((8, BN), lambda i: (0, i)),
        out_shape=jax.ShapeDtypeStruct((8, n), jnp.float32),
    )(enc, W0, W1, W2)

    mask = out8[7] > 0.0
    d_xyz = out8[0:3].T
    d_rot = out8[3:7].T
    return (mask, d_xyz, d_rot)
